# baseline (speedup 1.0000x reference)
import functools
import jax
import jax.numpy as jnp
from jax import lax
from jax.experimental import pallas as pl
from jax.experimental.pallas import tpu as pltpu
from jax.experimental.pallas import tpu_sc as plsc

_N = 10000
_D = 128
_H = 256
_NP = 10240
_RB = 1024
_NBLK = _NP // _RB
_HALF = _NP // 2
_DUMMY = _HALF
_ACC_ROWS = _HALF + 128
_E = 320000
_NTILES = 16
_NWORK = 2 * _NTILES
_RPW = _NP // _NWORK
_SB = 2048
_EB = 64
_NSB = -(-_E // _SB)
_EPAD = _NSB * _SB


def _ln(x, g, b):
    m = jnp.mean(x, axis=-1, keepdims=True)
    v = jnp.mean((x - m) * (x - m), axis=-1, keepdims=True)
    return (x - m) * lax.rsqrt(v + 1e-5) * g + b


def _mlp_body(nf, em, W0, b0, g0, be0, W1, b1, g1, be1, W2, b2, g2, be2,
              wa, ba, h_out, s1_out, s2_out):
    x = nf[...] + em[...]
    h = jnp.maximum(_ln(jnp.dot(x, W0[...], preferred_element_type=jnp.float32,
                                precision=lax.Precision.HIGHEST) + b0[...], g0[...], be0[...]), 0.0)
    h = jnp.maximum(_ln(jnp.dot(h, W1[...], preferred_element_type=jnp.float32,
                                precision=lax.Precision.HIGHEST) + b1[...], g1[...], be1[...]), 0.0)
    h = jnp.maximum(_ln(jnp.dot(h, W2[...], preferred_element_type=jnp.float32,
                                precision=lax.Precision.HIGHEST) + b2[...], g2[...], be2[...]), 0.0)
    h_out[...] = h
    s = jnp.dot(h, wa[...], preferred_element_type=jnp.float32,
                precision=lax.Precision.HIGHEST)
    s1_out[...] = jnp.reshape(s[:, 0], (_RB // 128, 128)) + ba[...]
    s2_out[...] = jnp.reshape(s[:, 1], (_RB // 128, 128))


def _mlp_stack(nf, em, W0, b0, g0, be0, W1, b1, g1, be1, W2, b2, g2, be2, wa, ba):
    row_spec = lambda w: pl.BlockSpec((_RB, w), lambda i: (i, 0))
    full = lambda a: pl.BlockSpec(a.shape, lambda i: (0,) * a.ndim)
    h, s1, s2 = pl.pallas_call(
        _mlp_body,
        grid=(_NBLK,),
        in_specs=[row_spec(_D), row_spec(_D),
                  full(W0), full(b0), full(g0), full(be0),
                  full(W1), full(b1), full(g1), full(be1),
                  full(W2), full(b2), full(g2), full(be2),
                  full(wa), full(ba)],
        out_specs=[pl.BlockSpec((_RB, _H), lambda i: (i, 0)),
                   pl.BlockSpec((_RB // 128, 128), lambda i: (i, 0)),
                   pl.BlockSpec((_RB // 128, 128), lambda i: (i, 0))],
        out_shape=[jax.ShapeDtypeStruct((_NP, _H), jnp.float32),
                   jax.ShapeDtypeStruct((_NP // 128, 128), jnp.float32),
                   jax.ShapeDtypeStruct((_NP // 128, 128), jnp.float32)],
    )(nf, em, W0, b0, g0, be0, W1, b1, g1, be1, W2, b2, g2, be2, wa, ba)
    return h, s1.reshape(_NP), s2.reshape(_NP)


def _edge_body(h_hbm, row_hbm, col_hbm, s1_hbm, s2_hbm, out_hbm,
               hbuf, rowv, colv, eidc, rowc, colc, av, s1t, s2t, acc2d, sem):
    c = lax.axis_index("c")
    t = lax.axis_index("s")
    wid = c * _NTILES + t
    lo = wid * _RPW
    iota16 = lax.iota(jnp.int32, 16)
    zero16 = jnp.zeros((16,), jnp.float32)

    pltpu.sync_copy(s1_hbm, s1t)
    pltpu.sync_copy(s2_hbm, s2t)

    def _z(i, carry):
        for q in range(_H // 16):
            acc2d[i, pl.ds(q * 16, 16)] = zero16
        return carry
    lax.fori_loop(0, _RPW, _z, None)

    def _scan_block(sb, carry):
        base = sb * _SB
        pltpu.sync_copy(row_hbm.at[pl.ds(base, _SB)], rowv)
        pltpu.sync_copy(col_hbm.at[pl.ds(base, _SB)], colv)

        def _group(g, cnt):
            rv = rowv[pl.ds(g * 16, 16)]
            m = (rv >= lo) & (rv < lo + _RPW)
            plsc.store_compressed(eidc.at[pl.ds(cnt, 16)], g * 16 + iota16,
                                  mask=m)
            return cnt + plsc.all_reduce_population_count(m)[0]
        cnt = lax.fori_loop(0, _SB // 16, _group, 0)

        def _batch(bb, carry2):
            ebase = bb * _EB
            for j in range(_EB // 16):
                el = eidc[pl.ds(ebase + j * 16, 16)] & (_SB - 1)
                rv = plsc.load_gather(rowv, [el])
                cv = plsc.load_gather(colv, [el])
                sv = plsc.load_gather(s1t, [rv]) + plsc.load_gather(s2t, [cv])
                a = 1.0 / (1.0 + jnp.exp(-sv))
                valid = (ebase + j * 16 + iota16) < cnt
                av[pl.ds(j * 16, 16)] = jnp.where(valid, a, 0.0)
                rowc[pl.ds(j * 16, 16)] = rv - lo
                colc[pl.ds(j * 16, 16)] = cv
            pltpu.async_copy(h_hbm.at[colc], hbuf, sem).wait()

            def _edge_group(g, carry3):
                base = g * 16
                rl16 = rowc[pl.ds(base, 16)]
                a16 = av[pl.ds(base, 16)]
                rl16 = jnp.minimum(jnp.maximum(rl16, 0), _RPW - 1)
                for l in range(16):
                    rl_s = rl16[l]
                    ab = lax.broadcast(a16[l], (16,))
                    e = base + l
                    for k in range(_H // 16):
                        v = hbuf[e, pl.ds(k * 16, 16)] * ab
                        acc2d[rl_s, pl.ds(k * 16, 16)] = (
                            acc2d[rl_s, pl.ds(k * 16, 16)] + v)
                return carry3
            lax.fori_loop(0, _EB // 16, _edge_group, None)
            return carry2
        nb = (cnt + _EB - 1) // _EB
        lax.fori_loop(0, nb, _batch, None)
        return carry
    lax.fori_loop(0, _NSB, _scan_block, None)

    pltpu.sync_copy(acc2d, out_hbm.at[pl.ds(lo, _RPW)])


@functools.cache
def _edge_kernel():
  return pl.kernel(
    _edge_body,
    out_type=jax.ShapeDtypeStruct((_NP, _H), jnp.float32),
    mesh=plsc.VectorSubcoreMesh(core_axis_name="c", subcore_axis_name="s",
                                num_cores=2, num_subcores=_NTILES),
    compiler_params=pltpu.CompilerParams(needs_layout_passes=False),
    scratch_types=[
        pltpu.VMEM((_EB, _H), jnp.float32),
        pltpu.VMEM((_SB,), jnp.int32),
        pltpu.VMEM((_SB,), jnp.int32),
        pltpu.VMEM((_SB + 16,), jnp.int32),
        pltpu.VMEM((_EB,), jnp.int32),
        pltpu.VMEM((_EB,), jnp.int32),
        pltpu.VMEM((_EB,), jnp.float32),
        pltpu.VMEM((_NP,), jnp.float32),
        pltpu.VMEM((_NP,), jnp.float32),
        pltpu.VMEM((_RPW, _H), jnp.float32),
        pltpu.SemaphoreType.DMA,
    ],
  )


def _final_body(h, hagg, W3, b3, g3, be3, out):
    z = h[...] + 0.5 * hagg[...]
    out[...] = _ln(jnp.dot(z, W3[...], preferred_element_type=jnp.float32,
                           precision=lax.Precision.HIGHEST) + b3[...], g3[...], be3[...])


def _final_layer(h, hagg, W3, b3, g3, be3):
    full = lambda a: pl.BlockSpec(a.shape, lambda i: (0,) * a.ndim)
    return pl.pallas_call(
        _final_body,
        grid=(_NBLK,),
        in_specs=[pl.BlockSpec((_RB, _H), lambda i: (i, 0)),
                  pl.BlockSpec((_RB, _H), lambda i: (i, 0)),
                  full(W3), full(b3), full(g3), full(be3)],
        out_specs=pl.BlockSpec((_RB, _D), lambda i: (i, 0)),
        out_shape=jax.ShapeDtypeStruct((_NP, _D), jnp.float32),
    )(h, hagg, W3, b3, g3, be3)


def kernel(node_ids, edge_index, node_features, emb, W0, b0, g0, be0,
           W1, b1, g1, be1, W2, b2, g2, be2, W3, b3, g3, be3, Wa, ba):
    del node_ids
    padn = ((0, _NP - _N), (0, 0))
    nf = jnp.pad(node_features, padn)
    em = jnp.pad(emb, padn)
    wa = jnp.stack([Wa[:_H, 0], Wa[_H:, 0]], axis=1)
    h, s1, s2 = _mlp_stack(
        nf, em, W0, b0.reshape(1, _H), g0.reshape(1, _H), be0.reshape(1, _H),
        W1, b1.reshape(1, _H), g1.reshape(1, _H), be1.reshape(1, _H),
        W2, b2.reshape(1, _H), g2.reshape(1, _H), be2.reshape(1, _H),
        wa, ba.reshape(1, 1))
    row = jnp.pad(edge_index[0], (0, _EPAD - _E), constant_values=_NP)
    col = jnp.pad(edge_index[1], (0, _EPAD - _E), constant_values=0)
    hagg = _edge_kernel()(h, row, col, s1, s2)
    out = _final_layer(h, hagg, W3, b3.reshape(1, _D), g3.reshape(1, _D),
                       be3.reshape(1, _D))
    return out[:_N]


# speedup vs baseline: 1.4767x; 1.4767x over previous
import functools
import jax
import jax.numpy as jnp
from jax import lax
from jax.experimental import pallas as pl
from jax.experimental.pallas import tpu as pltpu
from jax.experimental.pallas import tpu_sc as plsc

_N = 10000
_D = 128
_H = 256
_NP = 10240
_RB = 1024
_NBLK = _NP // _RB
_HALF = _NP // 2
_DUMMY = _HALF
_ACC_ROWS = _HALF + 128
_E = 320000
_NTILES = 16
_NWORK = 2 * _NTILES
_RPW = _NP // _NWORK
_SB = 2048
_EB = 64
_NSB = (-(-_E // _SB) + 1) // 2 * 2
_EPAD = _NSB * _SB
_EALLOC = _EPAD + 2 * _SB


def _ln(x, g, b):
    m = jnp.mean(x, axis=-1, keepdims=True)
    v = jnp.mean((x - m) * (x - m), axis=-1, keepdims=True)
    return (x - m) * lax.rsqrt(v + 1e-5) * g + b


def _mlp_body(nf, em, W0, b0, g0, be0, W1, b1, g1, be1, W2, b2, g2, be2,
              wa, ba, h_out, s1_out, s2_out):
    x = nf[...] + em[...]
    h = jnp.maximum(_ln(jnp.dot(x, W0[...], preferred_element_type=jnp.float32,
                                precision=lax.Precision.HIGHEST) + b0[...], g0[...], be0[...]), 0.0)
    h = jnp.maximum(_ln(jnp.dot(h, W1[...], preferred_element_type=jnp.float32,
                                precision=lax.Precision.HIGHEST) + b1[...], g1[...], be1[...]), 0.0)
    h = jnp.maximum(_ln(jnp.dot(h, W2[...], preferred_element_type=jnp.float32,
                                precision=lax.Precision.HIGHEST) + b2[...], g2[...], be2[...]), 0.0)
    h_out[...] = h
    s = jnp.dot(h, wa[...], preferred_element_type=jnp.float32,
                precision=lax.Precision.HIGHEST)
    s1_out[...] = jnp.reshape(s[:, 0], (_RB // 128, 128)) + ba[...]
    s2_out[...] = jnp.reshape(s[:, 1], (_RB // 128, 128))


def _mlp_stack(nf, em, W0, b0, g0, be0, W1, b1, g1, be1, W2, b2, g2, be2, wa, ba):
    row_spec = lambda w: pl.BlockSpec((_RB, w), lambda i: (i, 0))
    full = lambda a: pl.BlockSpec(a.shape, lambda i: (0,) * a.ndim)
    h, s1, s2 = pl.pallas_call(
        _mlp_body,
        grid=(_NBLK,),
        in_specs=[row_spec(_D), row_spec(_D),
                  full(W0), full(b0), full(g0), full(be0),
                  full(W1), full(b1), full(g1), full(be1),
                  full(W2), full(b2), full(g2), full(be2),
                  full(wa), full(ba)],
        out_specs=[pl.BlockSpec((_RB, _H), lambda i: (i, 0)),
                   pl.BlockSpec((_RB // 128, 128), lambda i: (i, 0)),
                   pl.BlockSpec((_RB // 128, 128), lambda i: (i, 0))],
        out_shape=[jax.ShapeDtypeStruct((_NP, _H), jnp.float32),
                   jax.ShapeDtypeStruct((_NP // 128, 128), jnp.float32),
                   jax.ShapeDtypeStruct((_NP // 128, 128), jnp.float32)],
    )(nf, em, W0, b0, g0, be0, W1, b1, g1, be1, W2, b2, g2, be2, wa, ba)
    return h, s1.reshape(_NP), s2.reshape(_NP)


def _edge_body(h_hbm, row_hbm, col_hbm, s1_hbm, s2_hbm, out_hbm,
               hbuf, rowva, colva, rowvb, colvb, eidc, rowc, colc, av,
               s1t, s2t, accf, semra, semca, semrb, semcb, sem):
    c = lax.axis_index("c")
    t = lax.axis_index("s")
    wid = c * _NTILES + t
    lo = wid * _RPW
    iota16 = lax.iota(jnp.int32, 16)
    zero16 = jnp.zeros((16,), jnp.float32)

    pltpu.sync_copy(s1_hbm, s1t)
    pltpu.sync_copy(s2_hbm, s2t)

    def _z(i, carry):
        for q in range(8):
            accf[pl.ds(i * 128 + q * 16, 16)] = zero16
        return carry
    lax.fori_loop(0, _RPW * _H // 128, _z, None)

    def _stage(sb, rowv, colv, semr, semc):
        base = sb * _SB
        pltpu.async_copy(row_hbm.at[pl.ds(base, _SB)], rowv, semr)
        pltpu.async_copy(col_hbm.at[pl.ds(base, _SB)], colv, semc)

    def _process(sb, rowv, colv, semr, semc):
        pltpu.make_async_copy(row_hbm.at[pl.ds(sb * _SB, _SB)], rowv,
                              semr).wait()
        pltpu.make_async_copy(col_hbm.at[pl.ds(sb * _SB, _SB)], colv,
                              semc).wait()

        def _scan4(q4, cnt):
            g0 = q4 * 4
            ms, pcs = [], []
            for dq in range(4):
                rv = rowv[pl.ds((g0 + dq) * 16, 16)]
                m = (rv >= lo) & (rv < lo + _RPW)
                ms.append(m)
                pcs.append(plsc.all_reduce_population_count(m)[0])
            offs = cnt
            for dq in range(4):
                plsc.store_compressed(eidc.at[pl.ds(offs, 16)],
                                      (g0 + dq) * 16 + iota16, mask=ms[dq])
                offs = offs + pcs[dq]
            return offs
        cnt = lax.fori_loop(0, _SB // 64, _scan4, 0)

        def _batch(bb, carry2):
            ebase = bb * _EB
            for j in range(_EB // 16):
                el = eidc[pl.ds(ebase + j * 16, 16)] & (_SB - 1)
                rv = plsc.load_gather(rowv, [el])
                cv = plsc.load_gather(colv, [el])
                sv = plsc.load_gather(s1t, [rv]) + plsc.load_gather(s2t, [cv])
                a = 1.0 / (1.0 + jnp.exp(-sv))
                valid = (ebase + j * 16 + iota16) < cnt
                av[pl.ds(j * 16, 16)] = jnp.where(valid, a, 0.0)
                rowc[pl.ds(j * 16, 16)] = rv - lo
                colc[pl.ds(j * 16, 16)] = cv
            pltpu.async_copy(h_hbm.at[colc], hbuf, sem).wait()

            def _edge(e, carry3):
                eb16 = lax.broadcast(e, (16,))
                rlb = plsc.load_gather(rowc, [eb16])
                ab = plsc.load_gather(av, [eb16])
                rlc = jnp.minimum(jnp.maximum(rlb, 0), _RPW - 1)
                idx = rlc * _H + iota16
                for k in range(_H // 16):
                    v = hbuf[e, pl.ds(k * 16, 16)] * ab
                    plsc.addupdate_scatter(accf, [idx + (k * 16)], v)
                return carry3
            lax.fori_loop(0, _EB, _edge, None)
            return carry2
        nb = (cnt + _EB - 1) // _EB
        lax.fori_loop(0, nb, _batch, None)

    _stage(0, rowva, colva, semra, semca)
    _stage(1, rowvb, colvb, semrb, semcb)

    def _pair(i, carry):
        sba = 2 * i
        _process(sba, rowva, colva, semra, semca)
        _stage(sba + 2, rowva, colva, semra, semca)
        _process(sba + 1, rowvb, colvb, semrb, semcb)
        _stage(sba + 3, rowvb, colvb, semrb, semcb)
        return carry
    lax.fori_loop(0, _NSB // 2, _pair, None)
    pltpu.make_async_copy(row_hbm.at[pl.ds(0, _SB)], rowva, semra).wait()
    pltpu.make_async_copy(col_hbm.at[pl.ds(0, _SB)], colva, semca).wait()
    pltpu.make_async_copy(row_hbm.at[pl.ds(0, _SB)], rowvb, semrb).wait()
    pltpu.make_async_copy(col_hbm.at[pl.ds(0, _SB)], colvb, semcb).wait()

    pltpu.sync_copy(accf, out_hbm.at[pl.ds(lo * _H, _RPW * _H)])


@functools.cache
def _edge_kernel():
  return pl.kernel(
    _edge_body,
    out_type=jax.ShapeDtypeStruct((_NP * _H,), jnp.float32),
    mesh=plsc.VectorSubcoreMesh(core_axis_name="c", subcore_axis_name="s",
                                num_cores=2, num_subcores=_NTILES),
    compiler_params=pltpu.CompilerParams(needs_layout_passes=False),
    scratch_types=[
        pltpu.VMEM((_EB, _H), jnp.float32),
        pltpu.VMEM((_SB,), jnp.int32),
        pltpu.VMEM((_SB,), jnp.int32),
        pltpu.VMEM((_SB,), jnp.int32),
        pltpu.VMEM((_SB,), jnp.int32),
        pltpu.VMEM((_SB + 16,), jnp.int32),
        pltpu.VMEM((_EB,), jnp.int32),
        pltpu.VMEM((_EB,), jnp.int32),
        pltpu.VMEM((_EB,), jnp.float32),
        pltpu.VMEM((_NP,), jnp.float32),
        pltpu.VMEM((_NP,), jnp.float32),
        pltpu.VMEM((_RPW * _H,), jnp.float32),
        pltpu.SemaphoreType.DMA,
        pltpu.SemaphoreType.DMA,
        pltpu.SemaphoreType.DMA,
        pltpu.SemaphoreType.DMA,
        pltpu.SemaphoreType.DMA,
    ],
  )


def _final_body(h, hagg, W3, b3, g3, be3, out):
    z = h[...] + 0.5 * hagg[...]
    out[...] = _ln(jnp.dot(z, W3[...], preferred_element_type=jnp.float32,
                           precision=lax.Precision.HIGHEST) + b3[...], g3[...], be3[...])


def _final_layer(h, hagg, W3, b3, g3, be3):
    full = lambda a: pl.BlockSpec(a.shape, lambda i: (0,) * a.ndim)
    return pl.pallas_call(
        _final_body,
        grid=(_NBLK,),
        in_specs=[pl.BlockSpec((_RB, _H), lambda i: (i, 0)),
                  pl.BlockSpec((_RB, _H), lambda i: (i, 0)),
                  full(W3), full(b3), full(g3), full(be3)],
        out_specs=pl.BlockSpec((_RB, _D), lambda i: (i, 0)),
        out_shape=jax.ShapeDtypeStruct((_NP, _D), jnp.float32),
    )(h, hagg, W3, b3, g3, be3)


def kernel(node_ids, edge_index, node_features, emb, W0, b0, g0, be0,
           W1, b1, g1, be1, W2, b2, g2, be2, W3, b3, g3, be3, Wa, ba):
    del node_ids
    padn = ((0, _NP - _N), (0, 0))
    nf = jnp.pad(node_features, padn)
    em = jnp.pad(emb, padn)
    wa = jnp.stack([Wa[:_H, 0], Wa[_H:, 0]], axis=1)
    h, s1, s2 = _mlp_stack(
        nf, em, W0, b0.reshape(1, _H), g0.reshape(1, _H), be0.reshape(1, _H),
        W1, b1.reshape(1, _H), g1.reshape(1, _H), be1.reshape(1, _H),
        W2, b2.reshape(1, _H), g2.reshape(1, _H), be2.reshape(1, _H),
        wa, ba.reshape(1, 1))
    row = jnp.pad(edge_index[0], (0, _EALLOC - _E), constant_values=_NP)
    col = jnp.pad(edge_index[1], (0, _EALLOC - _E), constant_values=0)
    hagg = _edge_kernel()(h, row, col, s1, s2).reshape(_NP, _H)
    out = _final_layer(h, hagg, W3, b3.reshape(1, _D), g3.reshape(1, _D),
                       be3.reshape(1, _D))
    return out[:_N]


# speedup vs baseline: 1.8803x; 1.2733x over previous
import functools
import jax
import jax.numpy as jnp
from jax import lax
from jax.experimental import pallas as pl
from jax.experimental.pallas import tpu as pltpu
from jax.experimental.pallas import tpu_sc as plsc

_N = 10000
_D = 128
_H = 256
_NP = 10240
_RB = 1024
_NBLK = _NP // _RB
_HALF = _NP // 2
_DUMMY = _HALF
_ACC_ROWS = _HALF + 128
_E = 320000
_NTILES = 16
_NWORK = 2 * _NTILES
_RPW = _NP // _NWORK
_SB = 2048
_EB = 32
_NSB = (-(-_E // _SB) + 1) // 2 * 2
_EPAD = _NSB * _SB
_EALLOC = _EPAD + 2 * _SB


def _ln(x, g, b):
    m = jnp.mean(x, axis=-1, keepdims=True)
    v = jnp.mean((x - m) * (x - m), axis=-1, keepdims=True)
    return (x - m) * lax.rsqrt(v + 1e-5) * g + b


def _mlp_body(nf, em, W0, b0, g0, be0, W1, b1, g1, be1, W2, b2, g2, be2,
              wa, ba, h_out, s1_out, s2_out):
    x = nf[...] + em[...]
    h = jnp.maximum(_ln(jnp.dot(x, W0[...], preferred_element_type=jnp.float32,
                                precision=lax.Precision.HIGHEST) + b0[...], g0[...], be0[...]), 0.0)
    h = jnp.maximum(_ln(jnp.dot(h, W1[...], preferred_element_type=jnp.float32,
                                precision=lax.Precision.HIGHEST) + b1[...], g1[...], be1[...]), 0.0)
    h = jnp.maximum(_ln(jnp.dot(h, W2[...], preferred_element_type=jnp.float32,
                                precision=lax.Precision.HIGHEST) + b2[...], g2[...], be2[...]), 0.0)
    h_out[...] = h
    s = jnp.dot(h, wa[...], preferred_element_type=jnp.float32,
                precision=lax.Precision.HIGHEST)
    s1_out[...] = jnp.reshape(s[:, 0], (_RB // 128, 128)) + ba[...]
    s2_out[...] = jnp.reshape(s[:, 1], (_RB // 128, 128))


def _mlp_stack(nf, em, W0, b0, g0, be0, W1, b1, g1, be1, W2, b2, g2, be2, wa, ba):
    row_spec = lambda w: pl.BlockSpec((_RB, w), lambda i: (i, 0))
    full = lambda a: pl.BlockSpec(a.shape, lambda i: (0,) * a.ndim)
    h, s1, s2 = pl.pallas_call(
        _mlp_body,
        grid=(_NBLK,),
        in_specs=[row_spec(_D), row_spec(_D),
                  full(W0), full(b0), full(g0), full(be0),
                  full(W1), full(b1), full(g1), full(be1),
                  full(W2), full(b2), full(g2), full(be2),
                  full(wa), full(ba)],
        out_specs=[pl.BlockSpec((_RB, _H), lambda i: (i, 0)),
                   pl.BlockSpec((_RB // 128, 128), lambda i: (i, 0)),
                   pl.BlockSpec((_RB // 128, 128), lambda i: (i, 0))],
        out_shape=[jax.ShapeDtypeStruct((_NP, _H), jnp.float32),
                   jax.ShapeDtypeStruct((_NP // 128, 128), jnp.float32),
                   jax.ShapeDtypeStruct((_NP // 128, 128), jnp.float32)],
    )(nf, em, W0, b0, g0, be0, W1, b1, g1, be1, W2, b2, g2, be2, wa, ba)
    return h, s1.reshape(_NP), s2.reshape(_NP)


def _edge_body(h_hbm, row_hbm, col_hbm, s1_hbm, s2_hbm, out_hbm,
               hbuf, hbuf2, rowva, colva, rowvb, colvb, eidc, rowc, colc, av,
               rowc2, colc2, av2, s1t, s2t, accf,
               semra, semca, semrb, semcb, sem, sem2):
    c = lax.axis_index("c")
    t = lax.axis_index("s")
    wid = c * _NTILES + t
    lo = wid * _RPW
    iota16 = lax.iota(jnp.int32, 16)
    zero16 = jnp.zeros((16,), jnp.float32)

    pltpu.sync_copy(s1_hbm, s1t)
    pltpu.sync_copy(s2_hbm, s2t)

    def _z(i, carry):
        for q in range(8):
            accf[pl.ds(i * 128 + q * 16, 16)] = zero16
        return carry
    lax.fori_loop(0, _RPW * _H // 128, _z, None)

    def _stage(sb, rowv, colv, semr, semc):
        base = sb * _SB
        pltpu.async_copy(row_hbm.at[pl.ds(base, _SB)], rowv, semr)
        pltpu.async_copy(col_hbm.at[pl.ds(base, _SB)], colv, semc)

    def _process(sb, rowv, colv, semr, semc):
        pltpu.make_async_copy(row_hbm.at[pl.ds(sb * _SB, _SB)], rowv,
                              semr).wait()
        pltpu.make_async_copy(col_hbm.at[pl.ds(sb * _SB, _SB)], colv,
                              semc).wait()

        def _scan4(q4, cnt):
            g0 = q4 * 4
            ms, pcs = [], []
            for dq in range(4):
                rv = rowv[pl.ds((g0 + dq) * 16, 16)]
                m = (rv >= lo) & (rv < lo + _RPW)
                ms.append(m)
                pcs.append(plsc.all_reduce_population_count(m)[0])
            offs = cnt
            for dq in range(4):
                plsc.store_compressed(eidc.at[pl.ds(offs, 16)],
                                      (g0 + dq) * 16 + iota16, mask=ms[dq])
                offs = offs + pcs[dq]
            return offs
        cnt = lax.fori_loop(0, _SB // 64, _scan4, 0)

        def _fill(bb, rowc_x, colc_x, av_x):
            ebase = bb * _EB
            for j in range(_EB // 16):
                el = eidc[pl.ds(ebase + j * 16, 16)] & (_SB - 1)
                rv = plsc.load_gather(rowv, [el])
                cv = plsc.load_gather(colv, [el])
                sv = plsc.load_gather(s1t, [rv]) + plsc.load_gather(s2t, [cv])
                a = 1.0 / (1.0 + jnp.exp(-sv))
                valid = (ebase + j * 16 + iota16) < cnt
                av_x[pl.ds(j * 16, 16)] = jnp.where(valid, a, 0.0)
                rowc_x[pl.ds(j * 16, 16)] = rv - lo
                colc_x[pl.ds(j * 16, 16)] = cv

        def _accum(hbuf_x, rowc_x, av_x):
            def _edge(e, carry3):
                eb16 = lax.broadcast(e, (16,))
                rlb = plsc.load_gather(rowc_x, [eb16])
                ab = plsc.load_gather(av_x, [eb16])
                rlc = jnp.minimum(jnp.maximum(rlb, 0), _RPW - 1)
                idx = rlc * _H + iota16
                for k in range(_H // 16):
                    v = hbuf_x[e, pl.ds(k * 16, 16)] * ab
                    plsc.addupdate_scatter(accf, [idx + (k * 16)], v)
                return carry3
            lax.fori_loop(0, _EB, _edge, None)

        nb = (cnt + _EB - 1) // _EB

        @pl.when(nb > 0)
        def _():
            _fill(0, rowc, colc, av)
            pltpu.async_copy(h_hbm.at[colc], hbuf, sem)

            def _bpair(i, carry2):
                b1 = 2 * i + 1

                @pl.when(b1 < nb)
                def _():
                    _fill(b1, rowc2, colc2, av2)
                    pltpu.async_copy(h_hbm.at[colc2], hbuf2, sem2)
                pltpu.make_async_copy(h_hbm.at[colc], hbuf, sem).wait()
                _accum(hbuf, rowc, av)

                @pl.when(b1 < nb)
                def _():
                    @pl.when(b1 + 1 < nb)
                    def _():
                        _fill(b1 + 1, rowc, colc, av)
                        pltpu.async_copy(h_hbm.at[colc], hbuf, sem)
                    pltpu.make_async_copy(h_hbm.at[colc2], hbuf2, sem2).wait()
                    _accum(hbuf2, rowc2, av2)
                return carry2
            lax.fori_loop(0, (nb + 1) // 2, _bpair, None)

    _stage(0, rowva, colva, semra, semca)
    _stage(1, rowvb, colvb, semrb, semcb)

    def _pair(i, carry):
        sba = 2 * i
        _process(sba, rowva, colva, semra, semca)
        _stage(sba + 2, rowva, colva, semra, semca)
        _process(sba + 1, rowvb, colvb, semrb, semcb)
        _stage(sba + 3, rowvb, colvb, semrb, semcb)
        return carry
    lax.fori_loop(0, _NSB // 2, _pair, None)
    pltpu.make_async_copy(row_hbm.at[pl.ds(0, _SB)], rowva, semra).wait()
    pltpu.make_async_copy(col_hbm.at[pl.ds(0, _SB)], colva, semca).wait()
    pltpu.make_async_copy(row_hbm.at[pl.ds(0, _SB)], rowvb, semrb).wait()
    pltpu.make_async_copy(col_hbm.at[pl.ds(0, _SB)], colvb, semcb).wait()

    pltpu.sync_copy(accf, out_hbm.at[pl.ds(lo * _H, _RPW * _H)])


@functools.cache
def _edge_kernel():
  return pl.kernel(
    _edge_body,
    out_type=jax.ShapeDtypeStruct((_NP * _H,), jnp.float32),
    mesh=plsc.VectorSubcoreMesh(core_axis_name="c", subcore_axis_name="s",
                                num_cores=2, num_subcores=_NTILES),
    compiler_params=pltpu.CompilerParams(needs_layout_passes=False),
    scratch_types=[
        pltpu.VMEM((_EB, _H), jnp.float32),
        pltpu.VMEM((_EB, _H), jnp.float32),
        pltpu.VMEM((_SB,), jnp.int32),
        pltpu.VMEM((_SB,), jnp.int32),
        pltpu.VMEM((_SB,), jnp.int32),
        pltpu.VMEM((_SB,), jnp.int32),
        pltpu.VMEM((_SB + 16,), jnp.int32),
        pltpu.VMEM((_EB,), jnp.int32),
        pltpu.VMEM((_EB,), jnp.int32),
        pltpu.VMEM((_EB,), jnp.float32),
        pltpu.VMEM((_EB,), jnp.int32),
        pltpu.VMEM((_EB,), jnp.int32),
        pltpu.VMEM((_EB,), jnp.float32),
        pltpu.VMEM((_NP,), jnp.float32),
        pltpu.VMEM((_NP,), jnp.float32),
        pltpu.VMEM((_RPW * _H,), jnp.float32),
        pltpu.SemaphoreType.DMA,
        pltpu.SemaphoreType.DMA,
        pltpu.SemaphoreType.DMA,
        pltpu.SemaphoreType.DMA,
        pltpu.SemaphoreType.DMA,
        pltpu.SemaphoreType.DMA,
    ],
  )


def _final_body(h, hagg, W3, b3, g3, be3, out):
    z = h[...] + 0.5 * hagg[...]
    out[...] = _ln(jnp.dot(z, W3[...], preferred_element_type=jnp.float32,
                           precision=lax.Precision.HIGHEST) + b3[...], g3[...], be3[...])


def _final_layer(h, hagg, W3, b3, g3, be3):
    full = lambda a: pl.BlockSpec(a.shape, lambda i: (0,) * a.ndim)
    return pl.pallas_call(
        _final_body,
        grid=(_NBLK,),
        in_specs=[pl.BlockSpec((_RB, _H), lambda i: (i, 0)),
                  pl.BlockSpec((_RB, _H), lambda i: (i, 0)),
                  full(W3), full(b3), full(g3), full(be3)],
        out_specs=pl.BlockSpec((_RB, _D), lambda i: (i, 0)),
        out_shape=jax.ShapeDtypeStruct((_NP, _D), jnp.float32),
    )(h, hagg, W3, b3, g3, be3)


def kernel(node_ids, edge_index, node_features, emb, W0, b0, g0, be0,
           W1, b1, g1, be1, W2, b2, g2, be2, W3, b3, g3, be3, Wa, ba):
    del node_ids
    padn = ((0, _NP - _N), (0, 0))
    nf = jnp.pad(node_features, padn)
    em = jnp.pad(emb, padn)
    wa = jnp.stack([Wa[:_H, 0], Wa[_H:, 0]], axis=1)
    h, s1, s2 = _mlp_stack(
        nf, em, W0, b0.reshape(1, _H), g0.reshape(1, _H), be0.reshape(1, _H),
        W1, b1.reshape(1, _H), g1.reshape(1, _H), be1.reshape(1, _H),
        W2, b2.reshape(1, _H), g2.reshape(1, _H), be2.reshape(1, _H),
        wa, ba.reshape(1, 1))
    row = jnp.pad(edge_index[0], (0, _EALLOC - _E), constant_values=_NP)
    col = jnp.pad(edge_index[1], (0, _EALLOC - _E), constant_values=0)
    hagg = _edge_kernel()(h, row, col, s1, s2).reshape(_NP, _H)
    out = _final_layer(h, hagg, W3, b3.reshape(1, _D), g3.reshape(1, _D),
                       be3.reshape(1, _D))
    return out[:_N]


# speedup vs baseline: 1.9356x; 1.0294x over previous
import functools
import jax
import jax.numpy as jnp
from jax import lax
from jax.experimental import pallas as pl
from jax.experimental.pallas import tpu as pltpu
from jax.experimental.pallas import tpu_sc as plsc

_N = 10000
_D = 128
_H = 256
_NP = 10240
_RB = 1024
_NBLK = _NP // _RB
_HALF = _NP // 2
_DUMMY = _HALF
_ACC_ROWS = _HALF + 128
_E = 320000
_NTILES = 16
_NWORK = 2 * _NTILES
_RPW = _NP // _NWORK
_SB = 2048
_EB = 32
_NSB = (-(-_E // _SB) + 1) // 2 * 2
_EPAD = _NSB * _SB
_EALLOC = _EPAD + 2 * _SB


def _ln(x, g, b):
    m = jnp.mean(x, axis=-1, keepdims=True)
    v = jnp.mean((x - m) * (x - m), axis=-1, keepdims=True)
    return (x - m) * lax.rsqrt(v + 1e-5) * g + b


def _mlp_body(nf, em, W0, b0, g0, be0, W1, b1, g1, be1, W2, b2, g2, be2,
              wa, ba, h_out, s1_out, s2_out):
    x = nf[...] + em[...]
    h = jnp.maximum(_ln(jnp.dot(x, W0[...], preferred_element_type=jnp.float32,
                                precision=lax.Precision.HIGHEST) + b0[...], g0[...], be0[...]), 0.0)
    h = jnp.maximum(_ln(jnp.dot(h, W1[...], preferred_element_type=jnp.float32,
                                precision=lax.Precision.HIGHEST) + b1[...], g1[...], be1[...]), 0.0)
    h = jnp.maximum(_ln(jnp.dot(h, W2[...], preferred_element_type=jnp.float32,
                                precision=lax.Precision.HIGHEST) + b2[...], g2[...], be2[...]), 0.0)
    h_out[...] = h
    s = jnp.dot(h, wa[...], preferred_element_type=jnp.float32,
                precision=lax.Precision.HIGHEST)
    s1_out[...] = jnp.reshape(s[:, 0], (_RB // 128, 128)) + ba[...]
    s2_out[...] = jnp.reshape(s[:, 1], (_RB // 128, 128))


def _mlp_stack(nf, em, W0, b0, g0, be0, W1, b1, g1, be1, W2, b2, g2, be2, wa, ba):
    row_spec = lambda w: pl.BlockSpec((_RB, w), lambda i: (i, 0))
    full = lambda a: pl.BlockSpec(a.shape, lambda i: (0,) * a.ndim)
    h, s1, s2 = pl.pallas_call(
        _mlp_body,
        grid=(_NBLK,),
        in_specs=[row_spec(_D), row_spec(_D),
                  full(W0), full(b0), full(g0), full(be0),
                  full(W1), full(b1), full(g1), full(be1),
                  full(W2), full(b2), full(g2), full(be2),
                  full(wa), full(ba)],
        out_specs=[pl.BlockSpec((_RB, _H), lambda i: (i, 0)),
                   pl.BlockSpec((_RB // 128, 128), lambda i: (i, 0)),
                   pl.BlockSpec((_RB // 128, 128), lambda i: (i, 0))],
        out_shape=[jax.ShapeDtypeStruct((_NP, _H), jnp.float32),
                   jax.ShapeDtypeStruct((_NP // 128, 128), jnp.float32),
                   jax.ShapeDtypeStruct((_NP // 128, 128), jnp.float32)],
    )(nf, em, W0, b0, g0, be0, W1, b1, g1, be1, W2, b2, g2, be2, wa, ba)
    return h, s1.reshape(_NP), s2.reshape(_NP)


def _edge_body(h_hbm, row_hbm, col_hbm, s1_hbm, s2_hbm, out_hbm,
               hbuf, hbuf2, rowva, colva, rowvb, colvb, eidc, rowc, colc, av,
               rowc2, colc2, av2, s1t, s2t, accf,
               semra, semca, semrb, semcb, sem, sem2):
    c = lax.axis_index("c")
    t = lax.axis_index("s")
    wid = c * _NTILES + t
    lo = wid * _RPW
    iota16 = lax.iota(jnp.int32, 16)
    zero16 = jnp.zeros((16,), jnp.float32)

    pltpu.sync_copy(s1_hbm, s1t)
    pltpu.sync_copy(s2_hbm, s2t)

    def _z(i, carry):
        for q in range(8):
            accf[pl.ds(i * 128 + q * 16, 16)] = zero16
        return carry
    lax.fori_loop(0, _RPW * _H // 128, _z, None)

    def _stage(sb, rowv, colv, semr, semc):
        base = sb * _SB
        pltpu.async_copy(row_hbm.at[pl.ds(base, _SB)], rowv, semr)
        pltpu.async_copy(col_hbm.at[pl.ds(base, _SB)], colv, semc)

    def _process(sb, rowv, colv, semr, semc):
        pltpu.make_async_copy(row_hbm.at[pl.ds(sb * _SB, _SB)], rowv,
                              semr).wait()
        pltpu.make_async_copy(col_hbm.at[pl.ds(sb * _SB, _SB)], colv,
                              semc).wait()

        def _scan8(q8, cnt):
            g0 = q8 * 8
            ms, pcs = [], []
            for dq in range(8):
                rv = rowv[pl.ds((g0 + dq) * 16, 16)]
                m = (rv >= lo) & (rv < lo + _RPW)
                ms.append(m)
                pcs.append(plsc.all_reduce_population_count(m)[0])
            offs = cnt
            for dq in range(8):
                plsc.store_compressed(eidc.at[pl.ds(offs, 16)],
                                      (g0 + dq) * 16 + iota16, mask=ms[dq])
                offs = offs + pcs[dq]
            return offs
        cnt = lax.fori_loop(0, _SB // 128, _scan8, 0)

        def _fill(bb, rowc_x, colc_x, av_x):
            ebase = bb * _EB
            for j in range(_EB // 16):
                el = eidc[pl.ds(ebase + j * 16, 16)] & (_SB - 1)
                rv = plsc.load_gather(rowv, [el])
                cv = plsc.load_gather(colv, [el])
                sv = plsc.load_gather(s1t, [rv]) + plsc.load_gather(s2t, [cv])
                a = 1.0 / (1.0 + jnp.exp(-sv))
                valid = (ebase + j * 16 + iota16) < cnt
                av_x[pl.ds(j * 16, 16)] = jnp.where(valid, a, 0.0)
                rowc_x[pl.ds(j * 16, 16)] = rv - lo
                colc_x[pl.ds(j * 16, 16)] = cv

        def _accum(hbuf_x, rowc_x, av_x):
            def _edge(e, carry3):
                eb16 = lax.broadcast(e, (16,))
                rlb = plsc.load_gather(rowc_x, [eb16])
                ab = plsc.load_gather(av_x, [eb16])
                rlc = jnp.minimum(jnp.maximum(rlb, 0), _RPW - 1)
                idx = rlc * _H + iota16
                for k in range(_H // 16):
                    v = hbuf_x[e, pl.ds(k * 16, 16)] * ab
                    plsc.addupdate_scatter(accf, [idx + (k * 16)], v)
                return carry3
            lax.fori_loop(0, _EB, _edge, None)

        nb = (cnt + _EB - 1) // _EB

        @pl.when(nb > 0)
        def _():
            _fill(0, rowc, colc, av)
            pltpu.async_copy(h_hbm.at[colc], hbuf, sem)

            def _bpair(i, carry2):
                b1 = 2 * i + 1

                @pl.when(b1 < nb)
                def _():
                    _fill(b1, rowc2, colc2, av2)
                    pltpu.async_copy(h_hbm.at[colc2], hbuf2, sem2)
                pltpu.make_async_copy(h_hbm.at[colc], hbuf, sem).wait()
                _accum(hbuf, rowc, av)

                @pl.when(b1 < nb)
                def _():
                    @pl.when(b1 + 1 < nb)
                    def _():
                        _fill(b1 + 1, rowc, colc, av)
                        pltpu.async_copy(h_hbm.at[colc], hbuf, sem)
                    pltpu.make_async_copy(h_hbm.at[colc2], hbuf2, sem2).wait()
                    _accum(hbuf2, rowc2, av2)
                return carry2
            lax.fori_loop(0, (nb + 1) // 2, _bpair, None)

    _stage(0, rowva, colva, semra, semca)
    _stage(1, rowvb, colvb, semrb, semcb)

    def _pair(i, carry):
        sba = 2 * i
        _process(sba, rowva, colva, semra, semca)
        _stage(sba + 2, rowva, colva, semra, semca)
        _process(sba + 1, rowvb, colvb, semrb, semcb)
        _stage(sba + 3, rowvb, colvb, semrb, semcb)
        return carry
    lax.fori_loop(0, _NSB // 2, _pair, None)
    pltpu.make_async_copy(row_hbm.at[pl.ds(0, _SB)], rowva, semra).wait()
    pltpu.make_async_copy(col_hbm.at[pl.ds(0, _SB)], colva, semca).wait()
    pltpu.make_async_copy(row_hbm.at[pl.ds(0, _SB)], rowvb, semrb).wait()
    pltpu.make_async_copy(col_hbm.at[pl.ds(0, _SB)], colvb, semcb).wait()

    pltpu.sync_copy(accf, out_hbm.at[pl.ds(lo * _H, _RPW * _H)])


@functools.cache
def _edge_kernel():
  return pl.kernel(
    _edge_body,
    out_type=jax.ShapeDtypeStruct((_NP * _H,), jnp.float32),
    mesh=plsc.VectorSubcoreMesh(core_axis_name="c", subcore_axis_name="s",
                                num_cores=2, num_subcores=_NTILES),
    compiler_params=pltpu.CompilerParams(needs_layout_passes=False),
    scratch_types=[
        pltpu.VMEM((_EB, _H), jnp.float32),
        pltpu.VMEM((_EB, _H), jnp.float32),
        pltpu.VMEM((_SB,), jnp.int32),
        pltpu.VMEM((_SB,), jnp.int32),
        pltpu.VMEM((_SB,), jnp.int32),
        pltpu.VMEM((_SB,), jnp.int32),
        pltpu.VMEM((_SB + 16,), jnp.int32),
        pltpu.VMEM((_EB,), jnp.int32),
        pltpu.VMEM((_EB,), jnp.int32),
        pltpu.VMEM((_EB,), jnp.float32),
        pltpu.VMEM((_EB,), jnp.int32),
        pltpu.VMEM((_EB,), jnp.int32),
        pltpu.VMEM((_EB,), jnp.float32),
        pltpu.VMEM((_NP,), jnp.float32),
        pltpu.VMEM((_NP,), jnp.float32),
        pltpu.VMEM((_RPW * _H,), jnp.float32),
        pltpu.SemaphoreType.DMA,
        pltpu.SemaphoreType.DMA,
        pltpu.SemaphoreType.DMA,
        pltpu.SemaphoreType.DMA,
        pltpu.SemaphoreType.DMA,
        pltpu.SemaphoreType.DMA,
    ],
  )


def _final_body(h, hagg, W3, b3, g3, be3, out):
    z = h[...] + 0.5 * hagg[...]
    out[...] = _ln(jnp.dot(z, W3[...], preferred_element_type=jnp.float32,
                           precision=lax.Precision.HIGHEST) + b3[...], g3[...], be3[...])


def _final_layer(h, hagg, W3, b3, g3, be3):
    full = lambda a: pl.BlockSpec(a.shape, lambda i: (0,) * a.ndim)
    return pl.pallas_call(
        _final_body,
        grid=(_NBLK,),
        in_specs=[pl.BlockSpec((_RB, _H), lambda i: (i, 0)),
                  pl.BlockSpec((_RB, _H), lambda i: (i, 0)),
                  full(W3), full(b3), full(g3), full(be3)],
        out_specs=pl.BlockSpec((_RB, _D), lambda i: (i, 0)),
        out_shape=jax.ShapeDtypeStruct((_NP, _D), jnp.float32),
    )(h, hagg, W3, b3, g3, be3)


def kernel(node_ids, edge_index, node_features, emb, W0, b0, g0, be0,
           W1, b1, g1, be1, W2, b2, g2, be2, W3, b3, g3, be3, Wa, ba):
    del node_ids
    padn = ((0, _NP - _N), (0, 0))
    nf = jnp.pad(node_features, padn)
    em = jnp.pad(emb, padn)
    wa = jnp.stack([Wa[:_H, 0], Wa[_H:, 0]], axis=1)
    h, s1, s2 = _mlp_stack(
        nf, em, W0, b0.reshape(1, _H), g0.reshape(1, _H), be0.reshape(1, _H),
        W1, b1.reshape(1, _H), g1.reshape(1, _H), be1.reshape(1, _H),
        W2, b2.reshape(1, _H), g2.reshape(1, _H), be2.reshape(1, _H),
        wa, ba.reshape(1, 1))
    row = jnp.pad(edge_index[0], (0, _EALLOC - _E), constant_values=_NP)
    col = jnp.pad(edge_index[1], (0, _EALLOC - _E), constant_values=0)
    hagg = _edge_kernel()(h, row, col, s1, s2).reshape(_NP, _H)
    out = _final_layer(h, hagg, W3, b3.reshape(1, _D), g3.reshape(1, _D),
                       be3.reshape(1, _D))
    return out[:_N]


# speedup vs baseline: 2.3005x; 1.1885x over previous
import functools
import jax
import jax.numpy as jnp
from jax import lax
from jax.experimental import pallas as pl
from jax.experimental.pallas import tpu as pltpu
from jax.experimental.pallas import tpu_sc as plsc

_N = 10000
_D = 128
_H = 256
_NP = 10240
_RB = 1024
_NBLK = _NP // _RB
_HALF = _NP // 2
_DUMMY = _HALF
_ACC_ROWS = _HALF + 128
_E = 320000
_NTILES = 16
_NWORK = 2 * _NTILES
_RPW = _NP // _NWORK
_SB = 2048
_EB = 32
_NSB = (-(-_E // _SB) + 1) // 2 * 2
_EPAD = _NSB * _SB
_EALLOC = _EPAD + 2 * _SB


def _ln(x, g, b):
    m = jnp.mean(x, axis=-1, keepdims=True)
    v = jnp.mean((x - m) * (x - m), axis=-1, keepdims=True)
    return (x - m) * lax.rsqrt(v + 1e-5) * g + b


def _mlp_body(nf, em, W0, b0, g0, be0, W1, b1, g1, be1, W2, b2, g2, be2,
              wa, ba, h_out, s1_out, s2_out):
    x = nf[...] + em[...]
    h = jnp.maximum(_ln(jnp.dot(x, W0[...], preferred_element_type=jnp.float32,
                                precision=lax.Precision.HIGHEST) + b0[...], g0[...], be0[...]), 0.0)
    h = jnp.maximum(_ln(jnp.dot(h, W1[...], preferred_element_type=jnp.float32,
                                precision=lax.Precision.HIGHEST) + b1[...], g1[...], be1[...]), 0.0)
    h = jnp.maximum(_ln(jnp.dot(h, W2[...], preferred_element_type=jnp.float32,
                                precision=lax.Precision.HIGHEST) + b2[...], g2[...], be2[...]), 0.0)
    h_out[...] = h
    s = jnp.dot(h, wa[...], preferred_element_type=jnp.float32,
                precision=lax.Precision.HIGHEST)
    s1_out[...] = jnp.reshape(s[:, 0], (_RB // 128, 128)) + ba[...]
    s2_out[...] = jnp.reshape(s[:, 1], (_RB // 128, 128))


def _mlp_stack(nf, em, W0, b0, g0, be0, W1, b1, g1, be1, W2, b2, g2, be2, wa, ba):
    row_spec = lambda w: pl.BlockSpec((_RB, w), lambda i: (i, 0))
    full = lambda a: pl.BlockSpec(a.shape, lambda i: (0,) * a.ndim)
    h, s1, s2 = pl.pallas_call(
        _mlp_body,
        grid=(_NBLK,),
        in_specs=[row_spec(_D), row_spec(_D),
                  full(W0), full(b0), full(g0), full(be0),
                  full(W1), full(b1), full(g1), full(be1),
                  full(W2), full(b2), full(g2), full(be2),
                  full(wa), full(ba)],
        out_specs=[pl.BlockSpec((_RB, _H), lambda i: (i, 0)),
                   pl.BlockSpec((_RB // 128, 128), lambda i: (i, 0)),
                   pl.BlockSpec((_RB // 128, 128), lambda i: (i, 0))],
        out_shape=[jax.ShapeDtypeStruct((_NP, _H), jnp.float32),
                   jax.ShapeDtypeStruct((_NP // 128, 128), jnp.float32),
                   jax.ShapeDtypeStruct((_NP // 128, 128), jnp.float32)],
    )(nf, em, W0, b0, g0, be0, W1, b1, g1, be1, W2, b2, g2, be2, wa, ba)
    return h, s1.reshape(_NP), s2.reshape(_NP)


def _edge_body(h_hbm, row_hbm, col_hbm, s1_hbm, s2_hbm, out_hbm,
               hbuf, hbuf2, rowva, colva, rowvb, colvb, eidc, rowc, colc, av,
               rowc2, colc2, av2, s1t, s2t, accf,
               semra, semca, semrb, semcb, sem, sem2):
    c = lax.axis_index("c")
    t = lax.axis_index("s")
    wid = c * _NTILES + t
    lo = wid * _RPW
    iota16 = lax.iota(jnp.int32, 16)
    zero16 = jnp.zeros((16,), jnp.float32)

    pltpu.sync_copy(s1_hbm, s1t)
    pltpu.sync_copy(s2_hbm, s2t)

    def _z(i, carry):
        for q in range(8):
            accf[pl.ds(i * 128 + q * 16, 16)] = zero16
        return carry
    lax.fori_loop(0, _RPW * _H // 128, _z, None)

    def _stage(sb, rowv, colv, semr, semc):
        base = sb * _SB
        pltpu.async_copy(row_hbm.at[pl.ds(base, _SB)], rowv, semr)
        pltpu.async_copy(col_hbm.at[pl.ds(base, _SB)], colv, semc)

    def _process(sb, rowv, colv, semr, semc):
        pltpu.make_async_copy(row_hbm.at[pl.ds(sb * _SB, _SB)], rowv,
                              semr).wait()
        pltpu.make_async_copy(col_hbm.at[pl.ds(sb * _SB, _SB)], colv,
                              semc).wait()

        def _scan8(q8, cnt):
            g0 = q8 * 8
            ms, pcs = [], []
            for dq in range(8):
                rv = rowv[pl.ds((g0 + dq) * 16, 16)]
                m = (rv >= lo) & (rv < lo + _RPW)
                ms.append(m)
                pcs.append(plsc.all_reduce_population_count(m)[0])
            offs = cnt
            for dq in range(8):
                plsc.store_compressed(eidc.at[pl.ds(offs, 16)],
                                      (g0 + dq) * 16 + iota16, mask=ms[dq])
                offs = offs + pcs[dq]
            return offs
        cnt = lax.fori_loop(0, _SB // 128, _scan8, 0)

        def _fill(bb, rowc_x, colc_x, av_x):
            ebase = bb * _EB
            for j in range(_EB // 16):
                el = eidc[pl.ds(ebase + j * 16, 16)] & (_SB - 1)
                rv = plsc.load_gather(rowv, [el])
                cv = plsc.load_gather(colv, [el])
                sv = plsc.load_gather(s1t, [rv]) + plsc.load_gather(s2t, [cv])
                a = 1.0 / (1.0 + jnp.exp(-sv))
                valid = (ebase + j * 16 + iota16) < cnt
                av_x[pl.ds(j * 16, 16)] = jnp.where(valid, a, 0.0)
                rowc_x[pl.ds(j * 16, 16)] = rv - lo
                colc_x[pl.ds(j * 16, 16)] = cv

        def _accum(hbuf_x, rowc_x, av_x, n_e):
            def _edge(e, carry3):
                eb16 = lax.broadcast(e, (16,))
                rlb = plsc.load_gather(rowc_x, [eb16])
                ab = plsc.load_gather(av_x, [eb16])
                rlc = jnp.minimum(jnp.maximum(rlb, 0), _RPW - 1)
                idx = rlc * _H + iota16
                for k in range(_H // 16):
                    v = hbuf_x[e, pl.ds(k * 16, 16)] * ab
                    plsc.addupdate_scatter(accf, [idx + (k * 16)], v)
                return carry3
            lax.fori_loop(0, n_e, _edge, None)

        nb = (cnt + _EB - 1) // _EB

        @pl.when(nb > 0)
        def _():
            _fill(0, rowc, colc, av)
            pltpu.async_copy(h_hbm.at[colc], hbuf, sem)

            def _bpair(i, carry2):
                b1 = 2 * i + 1

                @pl.when(b1 < nb)
                def _():
                    _fill(b1, rowc2, colc2, av2)
                    pltpu.async_copy(h_hbm.at[colc2], hbuf2, sem2)
                pltpu.make_async_copy(h_hbm.at[colc], hbuf, sem).wait()
                _accum(hbuf, rowc, av, jnp.minimum(cnt - (b1 - 1) * _EB, _EB))

                @pl.when(b1 < nb)
                def _():
                    @pl.when(b1 + 1 < nb)
                    def _():
                        _fill(b1 + 1, rowc, colc, av)
                        pltpu.async_copy(h_hbm.at[colc], hbuf, sem)
                    pltpu.make_async_copy(h_hbm.at[colc2], hbuf2, sem2).wait()
                    _accum(hbuf2, rowc2, av2, jnp.minimum(cnt - b1 * _EB, _EB))
                return carry2
            lax.fori_loop(0, (nb + 1) // 2, _bpair, None)

    _stage(0, rowva, colva, semra, semca)
    _stage(1, rowvb, colvb, semrb, semcb)

    def _pair(i, carry):
        sba = 2 * i
        _process(sba, rowva, colva, semra, semca)
        _stage(sba + 2, rowva, colva, semra, semca)
        _process(sba + 1, rowvb, colvb, semrb, semcb)
        _stage(sba + 3, rowvb, colvb, semrb, semcb)
        return carry
    lax.fori_loop(0, _NSB // 2, _pair, None)
    pltpu.make_async_copy(row_hbm.at[pl.ds(0, _SB)], rowva, semra).wait()
    pltpu.make_async_copy(col_hbm.at[pl.ds(0, _SB)], colva, semca).wait()
    pltpu.make_async_copy(row_hbm.at[pl.ds(0, _SB)], rowvb, semrb).wait()
    pltpu.make_async_copy(col_hbm.at[pl.ds(0, _SB)], colvb, semcb).wait()

    pltpu.sync_copy(accf, out_hbm.at[pl.ds(lo * _H, _RPW * _H)])


@functools.cache
def _edge_kernel():
  return pl.kernel(
    _edge_body,
    out_type=jax.ShapeDtypeStruct((_NP * _H,), jnp.float32),
    mesh=plsc.VectorSubcoreMesh(core_axis_name="c", subcore_axis_name="s",
                                num_cores=2, num_subcores=_NTILES),
    compiler_params=pltpu.CompilerParams(needs_layout_passes=False),
    scratch_types=[
        pltpu.VMEM((_EB, _H), jnp.float32),
        pltpu.VMEM((_EB, _H), jnp.float32),
        pltpu.VMEM((_SB,), jnp.int32),
        pltpu.VMEM((_SB,), jnp.int32),
        pltpu.VMEM((_SB,), jnp.int32),
        pltpu.VMEM((_SB,), jnp.int32),
        pltpu.VMEM((_SB + 16,), jnp.int32),
        pltpu.VMEM((_EB,), jnp.int32),
        pltpu.VMEM((_EB,), jnp.int32),
        pltpu.VMEM((_EB,), jnp.float32),
        pltpu.VMEM((_EB,), jnp.int32),
        pltpu.VMEM((_EB,), jnp.int32),
        pltpu.VMEM((_EB,), jnp.float32),
        pltpu.VMEM((_NP,), jnp.float32),
        pltpu.VMEM((_NP,), jnp.float32),
        pltpu.VMEM((_RPW * _H,), jnp.float32),
        pltpu.SemaphoreType.DMA,
        pltpu.SemaphoreType.DMA,
        pltpu.SemaphoreType.DMA,
        pltpu.SemaphoreType.DMA,
        pltpu.SemaphoreType.DMA,
        pltpu.SemaphoreType.DMA,
    ],
  )


def _final_body(h, hagg, W3, b3, g3, be3, out):
    z = h[...] + 0.5 * hagg[...]
    out[...] = _ln(jnp.dot(z, W3[...], preferred_element_type=jnp.float32,
                           precision=lax.Precision.HIGHEST) + b3[...], g3[...], be3[...])


def _final_layer(h, hagg, W3, b3, g3, be3):
    full = lambda a: pl.BlockSpec(a.shape, lambda i: (0,) * a.ndim)
    return pl.pallas_call(
        _final_body,
        grid=(_NBLK,),
        in_specs=[pl.BlockSpec((_RB, _H), lambda i: (i, 0)),
                  pl.BlockSpec((_RB, _H), lambda i: (i, 0)),
                  full(W3), full(b3), full(g3), full(be3)],
        out_specs=pl.BlockSpec((_RB, _D), lambda i: (i, 0)),
        out_shape=jax.ShapeDtypeStruct((_NP, _D), jnp.float32),
    )(h, hagg, W3, b3, g3, be3)


def kernel(node_ids, edge_index, node_features, emb, W0, b0, g0, be0,
           W1, b1, g1, be1, W2, b2, g2, be2, W3, b3, g3, be3, Wa, ba):
    del node_ids
    padn = ((0, _NP - _N), (0, 0))
    nf = jnp.pad(node_features, padn)
    em = jnp.pad(emb, padn)
    wa = jnp.stack([Wa[:_H, 0], Wa[_H:, 0]], axis=1)
    h, s1, s2 = _mlp_stack(
        nf, em, W0, b0.reshape(1, _H), g0.reshape(1, _H), be0.reshape(1, _H),
        W1, b1.reshape(1, _H), g1.reshape(1, _H), be1.reshape(1, _H),
        W2, b2.reshape(1, _H), g2.reshape(1, _H), be2.reshape(1, _H),
        wa, ba.reshape(1, 1))
    row = jnp.pad(edge_index[0], (0, _EALLOC - _E), constant_values=_NP)
    col = jnp.pad(edge_index[1], (0, _EALLOC - _E), constant_values=0)
    hagg = _edge_kernel()(h, row, col, s1, s2).reshape(_NP, _H)
    out = _final_layer(h, hagg, W3, b3.reshape(1, _D), g3.reshape(1, _D),
                       be3.reshape(1, _D))
    return out[:_N]
